# baseline (speedup 1.0000x reference)
import functools

import jax
import jax.numpy as jnp
from jax import lax
from jax.experimental import pallas as pl
from jax.experimental.pallas import tpu as pltpu
from jax.experimental.pallas import tpu_sc as plsc

N = 10000
D = 128
C = 128
OUT = 128
E = 320000

NC = 2
NS = 16
NT = NC * NS

B = 64
BLOCKS = 160
EPT = B * BLOCKS
E_PAD = EPT * NT

ACC_ROWS = 10240
ROWS_PER_TILE = ACC_ROWS // NS
DUMMY = N + 8


def _sc_body(xl_hbm, xr_hbm, att_hbm, idx_hbm,
             out_msg, out_den,
             i0, i1, i2, i3, xlb0, xlb1, xrb0, xrb1, att_v,
             a_buf, den_buf, acc, is0, is1, is2, is3, gs0, gs1, ss0, ss1):
    idxs = (i0, i1, i2, i3)
    isems = (is0, is1, is2, is3)
    xls = (xlb0, xlb1)
    xrs = (xrb0, xrb1)
    gsems = (gs0, gs1)
    ssems = (ss0, ss1)

    cid = lax.axis_index("c")
    sid = lax.axis_index("s")
    wid = sid * NC + cid

    zeros16 = jnp.zeros((16,), jnp.float32)

    def zrow(i, carry):
        for c in range(C // 16):
            xrb0[i, pl.ds(c * 16, 16)] = zeros16
        return carry

    lax.fori_loop(0, B, zrow, 0)

    row0 = sid * ROWS_PER_TILE
    for k in range(ROWS_PER_TILE // B):
        pltpu.sync_copy(xrb0, acc.at[pl.ds(row0 + k * B, B)])
    plsc.subcore_barrier()

    def zden(i, carry):
        den_buf[pl.ds(i * 16, 16)] = zeros16
        return carry

    lax.fori_loop(0, ACC_ROWS // 16, zden, 0)

    pltpu.sync_copy(att_hbm, att_v)

    lane = lax.iota(jnp.int32, 16)

    def load_idx(r, slot):
        pltpu.async_copy(idx_hbm.at[wid, r], idxs[slot], isems[slot])

    def wait_idx(slot):
        pltpu.make_async_copy(idx_hbm.at[wid, 0], idxs[slot],
                              isems[slot]).wait()

    def start_gathers(b, slot):
        pltpu.async_copy(xl_hbm.at[idxs[slot].at[0]], xls[b], gsems[b])
        pltpu.async_copy(xr_hbm.at[idxs[slot].at[1]], xrs[b], gsems[b])

    def wait_gathers(b, slot):
        pltpu.make_async_copy(xl_hbm.at[idxs[slot].at[0]], xls[b],
                              gsems[b]).wait()
        pltpu.make_async_copy(xr_hbm.at[idxs[slot].at[1]], xrs[b],
                              gsems[b]).wait()

    def start_scatter(b, slot):
        pltpu.async_copy(xls[b], acc.at[idxs[slot].at[2]], ssems[b], add=True)

    def wait_scatter(b, slot):
        pltpu.make_async_copy(xls[b], acc.at[idxs[slot].at[2]],
                              ssems[b]).wait()

    def compute(b, slot):
        xlv = xls[b]
        xrv = xrs[b]
        idxv = idxs[slot]

        def grp_alpha(j, c2):
            rows = j * 16 + lane
            accs = [zeros16, zeros16, zeros16, zeros16]
            for c8 in range(C // 16):
                att16 = att_v[pl.ds(c8 * 16, 16)]
                for k in range(16):
                    colv = jnp.full((16,), c8 * 16 + k, jnp.int32)
                    xlg = plsc.load_gather(xlv, (rows, colv))
                    xrg = plsc.load_gather(xrv, (rows, colv))
                    m = xlg + xrg
                    lr = jnp.maximum(m, 0.2 * m)
                    accs[k % 4] = accs[k % 4] + lr * att16[k]
            acc16 = (accs[0] + accs[1]) + (accs[2] + accs[3])
            ex16 = jnp.exp(acc16)
            a_buf[pl.ds(j * 16, 16)] = ex16
            dst16 = idxv[2, pl.ds(j * 16, 16)]
            plsc.addupdate_scatter(den_buf, (dst16,), ex16)
            return c2

        lax.fori_loop(0, B // 16, grp_alpha, 0)

        def grp_msg(j, c2):
            ex16 = a_buf[pl.ds(j * 16, 16)]
            for k in range(16):
                i = j * 16 + k
                exs = ex16[k]
                for c in range(C // 16):
                    xlv[i, pl.ds(c * 16, 16)] = (
                        xlv[i, pl.ds(c * 16, 16)] * exs)
            return c2

        lax.fori_loop(0, B // 16, grp_msg, 0)

    load_idx(0, 0)
    load_idx(1, 1)
    wait_idx(0)
    start_gathers(0, 0)

    def quad(g, carry):
        for qq in range(4):
            r = g * 4 + qq
            b = qq % 2
            slot = qq
            nb = (qq + 1) % 2
            nslot = (qq + 1) % 4
            pslot = (qq + 2) % 4

            wait_gathers(b, slot)

            @pl.when(r + 1 < BLOCKS)
            def _():
                wait_idx(nslot)
                if qq == 0:
                    @pl.when(g > 0)
                    def _():
                        wait_scatter(nb, (qq + 3) % 4)
                else:
                    wait_scatter(nb, (qq + 3) % 4)
                start_gathers(nb, nslot)

            @pl.when(r + 2 < BLOCKS)
            def _():
                load_idx(r + 2, pslot)

            compute(b, slot)
            start_scatter(b, slot)
        return carry

    lax.fori_loop(0, BLOCKS // 4, quad, 0)

    wait_scatter(0, (BLOCKS - 2) % 4)
    wait_scatter(1, (BLOCKS - 1) % 4)

    plsc.subcore_barrier()
    pltpu.sync_copy(acc.at[pl.ds(row0, ROWS_PER_TILE)],
                    out_msg.at[cid, pl.ds(row0, ROWS_PER_TILE)])
    pltpu.sync_copy(den_buf, out_den.at[wid])


@functools.lru_cache(maxsize=None)
def _build_gat_pass():
  return pl.kernel(
    _sc_body,
    out_type=[jax.ShapeDtypeStruct((NC, ACC_ROWS, C), jnp.float32),
              jax.ShapeDtypeStruct((NT, ACC_ROWS), jnp.float32)],
    mesh=plsc.VectorSubcoreMesh(core_axis_name="c", subcore_axis_name="s",
                                num_cores=NC, num_subcores=NS),
    compiler_params=pltpu.CompilerParams(needs_layout_passes=False),
    scratch_types=[
        pltpu.VMEM((3, B), jnp.int32),
        pltpu.VMEM((3, B), jnp.int32),
        pltpu.VMEM((3, B), jnp.int32),
        pltpu.VMEM((3, B), jnp.int32),
        pltpu.VMEM((B, C), jnp.float32),
        pltpu.VMEM((B, C), jnp.float32),
        pltpu.VMEM((B, C), jnp.float32),
        pltpu.VMEM((B, C), jnp.float32),
        pltpu.VMEM((C,), jnp.float32),
        pltpu.VMEM((B,), jnp.float32),
        pltpu.VMEM((ACC_ROWS,), jnp.float32),
        pltpu.VMEM_SHARED((ACC_ROWS, C), jnp.float32),
        pltpu.SemaphoreType.DMA,
        pltpu.SemaphoreType.DMA,
        pltpu.SemaphoreType.DMA,
        pltpu.SemaphoreType.DMA,
        pltpu.SemaphoreType.DMA,
        pltpu.SemaphoreType.DMA,
        pltpu.SemaphoreType.DMA,
        pltpu.SemaphoreType.DMA,
    ],
  )


_RB = 1000
_RB2 = 1024


def _k1_body(x_ref, wl_ref, wr_ref, wcb_ref,
             xl0_ref, xl1_ref, xr0_ref, xr1_ref, xc_ref):
    xb = x_ref[...]
    l = jnp.dot(xb, wl_ref[...], preferred_element_type=jnp.float32)
    xl0_ref[...] = l[:, :C]
    xl1_ref[...] = l[:, C:]
    r = jnp.dot(xb, wr_ref[...], preferred_element_type=jnp.float32)
    xr0_ref[...] = r[:, :C]
    xr1_ref[...] = r[:, C:]
    xc_ref[...] = jnp.dot(xb, wcb_ref[...], preferred_element_type=jnp.float32)


def _k2_body(a0_ref, a1_ref, d0_ref, d1_ref, b1_ref, w2l_ref, w2r_ref,
             xl2_ref, xr2_ref):
    den0 = jnp.sum(d0_ref[...], axis=0).reshape(_RB2, 1)
    den1 = jnp.sum(d1_ref[...], axis=0).reshape(_RB2, 1)
    r0 = (a0_ref[0] + a0_ref[1]) / (den0 + 1e-16)
    r1 = (a1_ref[0] + a1_ref[1]) / (den1 + 1e-16)
    h = jnp.concatenate([r0, r1], axis=1) + b1_ref[...]
    h = jnp.where(h > 0, h, jnp.exp(h) - 1.0)
    xl2_ref[...] = jnp.dot(h, w2l_ref[...], preferred_element_type=jnp.float32)
    xr2_ref[...] = jnp.dot(h, w2r_ref[...], preferred_element_type=jnp.float32)


def _k3_body(a2_ref, d2_ref, b2_ref, wct_ref, xc_ref, bc_ref, out_ref):
    den = jnp.sum(d2_ref[...], axis=0).reshape(_RB2, 1)
    h2 = (a2_ref[0] + a2_ref[1]) / (den + 1e-16) + b2_ref[...]
    out_ref[...] = (jnp.dot(h2, wct_ref[...], preferred_element_type=jnp.float32)
                    + xc_ref[...] + bc_ref[...])


def _full_spec(shape):
    nd = len(shape)
    return pl.BlockSpec(shape, lambda i: (0,) * nd)


def kernel(x, edge_index, W1_l, W1_r, att1, b1, W2_l, W2_r, att2, b2, Wc, bc):
    src = edge_index[0].astype(jnp.int32)
    dst = edge_index[1].astype(jnp.int32)
    pad = E_PAD - E
    zpad = jnp.zeros((pad,), jnp.int32)
    src_p = jnp.concatenate([src, zpad])
    dstg_p = jnp.concatenate([dst, zpad])
    dsts_p = jnp.concatenate([dst, jnp.full((pad,), DUMMY, jnp.int32)])
    idx_all = (jnp.stack([src_p, dstg_p, dsts_p])
               .reshape(3, NT, BLOCKS, B).transpose(1, 2, 0, 3))

    f32 = jnp.float32

    xl0, xl1, xr0, xr1, xc = pl.pallas_call(
        _k1_body,
        grid=(N // _RB,),
        in_specs=[pl.BlockSpec((_RB, D), lambda i: (i, 0)),
                  _full_spec((D, 2 * C)), _full_spec((D, 2 * C)),
                  _full_spec((D, OUT))],
        out_specs=[pl.BlockSpec((_RB, C), lambda i: (i, 0))] * 4
        + [pl.BlockSpec((_RB, OUT), lambda i: (i, 0))],
        out_shape=[jax.ShapeDtypeStruct((N, C), f32)] * 4
        + [jax.ShapeDtypeStruct((N, OUT), f32)],
    )(x, W1_l, W1_r, Wc[D:])

    _gat_pass = _build_gat_pass()
    acc_h0, den_h0 = _gat_pass(xl0, xr0, att1[0], idx_all)
    acc_h1, den_h1 = _gat_pass(xl1, xr1, att1[1], idx_all)

    acc_spec = pl.BlockSpec((NC, _RB2, C), lambda i: (0, i, 0))
    den_spec = pl.BlockSpec((NT, _RB2), lambda i: (0, i))

    xl2f, xr2f = pl.pallas_call(
        _k2_body,
        grid=(ACC_ROWS // _RB2,),
        in_specs=[acc_spec, acc_spec, den_spec, den_spec,
                  _full_spec((1, 2 * C)), _full_spec((2 * C, C)),
                  _full_spec((2 * C, C))],
        out_specs=[pl.BlockSpec((_RB2, C), lambda i: (i, 0))] * 2,
        out_shape=[jax.ShapeDtypeStruct((ACC_ROWS, C), f32)] * 2,
    )(acc_h0, acc_h1, den_h0, den_h1, b1.reshape(1, 2 * C), W2_l, W2_r)

    acc2, den2 = _gat_pass(xl2f[:N], xr2f[:N], att2[0], idx_all)

    xc_pad = jnp.pad(xc, ((0, ACC_ROWS - N), (0, 0)))
    out_pad = pl.pallas_call(
        _k3_body,
        grid=(ACC_ROWS // _RB2,),
        in_specs=[acc_spec, den_spec, _full_spec((1, C)),
                  _full_spec((C, OUT)),
                  pl.BlockSpec((_RB2, OUT), lambda i: (i, 0)),
                  _full_spec((1, OUT))],
        out_specs=pl.BlockSpec((_RB2, OUT), lambda i: (i, 0)),
        out_shape=jax.ShapeDtypeStruct((ACC_ROWS, OUT), f32),
    )(acc2, den2, b2.reshape(1, C), Wc[:D], xc_pad, bc.reshape(1, OUT))

    return out_pad[:N]


# speedup vs baseline: 2.0603x; 2.0603x over previous
import functools

import jax
import jax.numpy as jnp
from jax import lax
from jax.experimental import pallas as pl
from jax.experimental.pallas import tpu as pltpu
from jax.experimental.pallas import tpu_sc as plsc

N = 10000
D = 128
C = 128
OUT = 128
E = 320000

NC = 2
NS = 16
NT = NC * NS

B = 64
BLOCKS = 160
EPT = B * BLOCKS
E_PAD = EPT * NT

ACC_ROWS = 10240
ROWS_PER_TILE = ACC_ROWS // NS
DUMMY = N + 8


def _sc_body(xl_hbm, xr_hbm, att_hbm, idx_hbm,
             out_msg, out_den,
             i0, i1, i2, i3, xlb0, xlb1, xrb0, xrb1, att_v,
             a_buf, den_buf, acc,
             is0, is1, is2, is3, gs0, gs1, ss0, ss1):
    idxs = (i0, i1, i2, i3)
    isems = (is0, is1, is2, is3)
    xls = (xlb0, xlb1)
    xrs = (xrb0, xrb1)
    gsems = (gs0, gs1)
    ssems = (ss0, ss1)

    cid = lax.axis_index("c")
    sid = lax.axis_index("s")
    wid = sid * NC + cid

    zeros16 = jnp.zeros((16,), jnp.float32)

    def zrow(i, carry):
        for c in range(C // 16):
            xrb0[i, pl.ds(c * 16, 16)] = zeros16
        return carry

    lax.fori_loop(0, B, zrow, 0)

    row0 = sid * ROWS_PER_TILE
    for k in range(ROWS_PER_TILE // B):
        pltpu.sync_copy(xrb0, acc.at[pl.ds(row0 + k * B, B)])
    plsc.subcore_barrier()

    def zden(i, carry):
        den_buf[pl.ds(i * 16, 16)] = zeros16
        return carry

    lax.fori_loop(0, ACC_ROWS // 16, zden, 0)

    pltpu.sync_copy(att_hbm, att_v)

    lane = lax.iota(jnp.int32, 16)

    def load_idx(r, slot):
        pltpu.async_copy(idx_hbm.at[wid, r], idxs[slot], isems[slot])

    def wait_idx(slot):
        pltpu.make_async_copy(idx_hbm.at[wid, 0], idxs[slot],
                              isems[slot]).wait()

    def start_gathers(b, slot):
        pltpu.async_copy(xl_hbm.at[idxs[slot].at[0]], xls[b], gsems[b])
        pltpu.async_copy(xr_hbm.at[idxs[slot].at[1]], xrs[b], gsems[b])

    def wait_gathers(b, slot):
        pltpu.make_async_copy(xl_hbm.at[idxs[slot].at[0]], xls[b],
                              gsems[b]).wait()
        pltpu.make_async_copy(xr_hbm.at[idxs[slot].at[1]], xrs[b],
                              gsems[b]).wait()

    def start_scatter(b, slot):
        pltpu.async_copy(xls[b], acc.at[idxs[slot].at[2]], ssems[b], add=True)

    def wait_scatter(b, slot):
        pltpu.make_async_copy(xls[b], acc.at[idxs[slot].at[2]],
                              ssems[b]).wait()

    def compute(b, slot):
        xlv = xls[b]
        xrv = xrs[b]
        idxv = idxs[slot]

        attc = [att_v[pl.ds(c8 * 16, 16)] for c8 in range(C // 16)]

        def edge_part(it, c2):
            for u in range(2):
                i = it * 2 + u
                pa = zeros16
                pb = zeros16
                for c8 in range(C // 16):
                    m = xlv[i, pl.ds(c8 * 16, 16)] + xrv[i, pl.ds(c8 * 16, 16)]
                    lr = jnp.maximum(m, 0.2 * m)
                    if c8 % 2 == 0:
                        pa = pa + lr * attc[c8]
                    else:
                        pb = pb + lr * attc[c8]
                xrv[i, pl.ds(0, 16)] = pa + pb
            return c2

        lax.fori_loop(0, B // 2, edge_part, 0)

        def grp_alpha(j, c2):
            rows = j * 16 + lane
            accs = [zeros16, zeros16, zeros16, zeros16]
            for k in range(16):
                colv = jnp.full((16,), k, jnp.int32)
                g = plsc.load_gather(xrv, (rows, colv))
                accs[k % 4] = accs[k % 4] + g
            alpha16 = (accs[0] + accs[1]) + (accs[2] + accs[3])
            ex16 = jnp.exp(alpha16)
            a_buf[pl.ds(j * 16, 16)] = ex16
            dst16 = idxv[2, pl.ds(j * 16, 16)]
            plsc.addupdate_scatter(den_buf, (dst16,), ex16)
            return c2

        lax.fori_loop(0, B // 16, grp_alpha, 0)

        def grp_msg(j, c2):
            ex16 = a_buf[pl.ds(j * 16, 16)]
            for k in range(16):
                i = j * 16 + k
                exs = ex16[k]
                for c in range(C // 16):
                    xlv[i, pl.ds(c * 16, 16)] = (
                        xlv[i, pl.ds(c * 16, 16)] * exs)
            return c2

        lax.fori_loop(0, B // 16, grp_msg, 0)

    load_idx(0, 0)
    load_idx(1, 1)
    wait_idx(0)
    start_gathers(0, 0)

    def quad(g, carry):
        for qq in range(4):
            r = g * 4 + qq
            b = qq % 2
            slot = qq
            nb = (qq + 1) % 2
            nslot = (qq + 1) % 4
            pslot = (qq + 2) % 4

            wait_gathers(b, slot)

            @pl.when(r + 1 < BLOCKS)
            def _():
                wait_idx(nslot)
                if qq == 0:
                    @pl.when(g > 0)
                    def _():
                        wait_scatter(nb, (qq + 3) % 4)
                else:
                    wait_scatter(nb, (qq + 3) % 4)
                start_gathers(nb, nslot)

            @pl.when(r + 2 < BLOCKS)
            def _():
                load_idx(r + 2, pslot)

            compute(b, slot)
            start_scatter(b, slot)
        return carry

    lax.fori_loop(0, BLOCKS // 4, quad, 0)

    wait_scatter(0, (BLOCKS - 2) % 4)
    wait_scatter(1, (BLOCKS - 1) % 4)

    plsc.subcore_barrier()
    pltpu.sync_copy(acc.at[pl.ds(row0, ROWS_PER_TILE)],
                    out_msg.at[cid, pl.ds(row0, ROWS_PER_TILE)])
    pltpu.sync_copy(den_buf, out_den.at[wid])


@functools.lru_cache(maxsize=None)
def _build_gat_pass():
  return pl.kernel(
    _sc_body,
    out_type=[jax.ShapeDtypeStruct((NC, ACC_ROWS, C), jnp.float32),
              jax.ShapeDtypeStruct((NT, ACC_ROWS), jnp.float32)],
    mesh=plsc.VectorSubcoreMesh(core_axis_name="c", subcore_axis_name="s",
                                num_cores=NC, num_subcores=NS),
    compiler_params=pltpu.CompilerParams(needs_layout_passes=False),
    scratch_types=[
        pltpu.VMEM((3, B), jnp.int32),
        pltpu.VMEM((3, B), jnp.int32),
        pltpu.VMEM((3, B), jnp.int32),
        pltpu.VMEM((3, B), jnp.int32),
        pltpu.VMEM((B, C), jnp.float32),
        pltpu.VMEM((B, C), jnp.float32),
        pltpu.VMEM((B, C), jnp.float32),
        pltpu.VMEM((B, C), jnp.float32),
        pltpu.VMEM((C,), jnp.float32),
        pltpu.VMEM((B,), jnp.float32),
        pltpu.VMEM((ACC_ROWS,), jnp.float32),
        pltpu.VMEM_SHARED((ACC_ROWS, C), jnp.float32),
        pltpu.SemaphoreType.DMA,
        pltpu.SemaphoreType.DMA,
        pltpu.SemaphoreType.DMA,
        pltpu.SemaphoreType.DMA,
        pltpu.SemaphoreType.DMA,
        pltpu.SemaphoreType.DMA,
        pltpu.SemaphoreType.DMA,
        pltpu.SemaphoreType.DMA,
    ],
  )


_RB = 1000
_RB2 = 1024


def _k1_body(x_ref, wl_ref, wr_ref, wcb_ref,
             xl0_ref, xl1_ref, xr0_ref, xr1_ref, xc_ref):
    xb = x_ref[...]
    l = jnp.dot(xb, wl_ref[...], preferred_element_type=jnp.float32)
    xl0_ref[...] = l[:, :C]
    xl1_ref[...] = l[:, C:]
    r = jnp.dot(xb, wr_ref[...], preferred_element_type=jnp.float32)
    xr0_ref[...] = r[:, :C]
    xr1_ref[...] = r[:, C:]
    xc_ref[...] = jnp.dot(xb, wcb_ref[...], preferred_element_type=jnp.float32)


def _k2_body(a0_ref, a1_ref, d0_ref, d1_ref, b1_ref, w2l_ref, w2r_ref,
             xl2_ref, xr2_ref):
    den0 = jnp.sum(d0_ref[...], axis=0).reshape(_RB2, 1)
    den1 = jnp.sum(d1_ref[...], axis=0).reshape(_RB2, 1)
    r0 = (a0_ref[0] + a0_ref[1]) / (den0 + 1e-16)
    r1 = (a1_ref[0] + a1_ref[1]) / (den1 + 1e-16)
    h = jnp.concatenate([r0, r1], axis=1) + b1_ref[...]
    h = jnp.where(h > 0, h, jnp.exp(h) - 1.0)
    xl2_ref[...] = jnp.dot(h, w2l_ref[...], preferred_element_type=jnp.float32)
    xr2_ref[...] = jnp.dot(h, w2r_ref[...], preferred_element_type=jnp.float32)


def _k3_body(a2_ref, d2_ref, b2_ref, wct_ref, xc_ref, bc_ref, out_ref):
    den = jnp.sum(d2_ref[...], axis=0).reshape(_RB2, 1)
    h2 = (a2_ref[0] + a2_ref[1]) / (den + 1e-16) + b2_ref[...]
    out_ref[...] = (jnp.dot(h2, wct_ref[...], preferred_element_type=jnp.float32)
                    + xc_ref[...] + bc_ref[...])


def _full_spec(shape):
    nd = len(shape)
    return pl.BlockSpec(shape, lambda i: (0,) * nd)


def kernel(x, edge_index, W1_l, W1_r, att1, b1, W2_l, W2_r, att2, b2, Wc, bc):
    src = edge_index[0].astype(jnp.int32)
    dst = edge_index[1].astype(jnp.int32)
    pad = E_PAD - E
    zpad = jnp.zeros((pad,), jnp.int32)
    src_p = jnp.concatenate([src, zpad])
    dstg_p = jnp.concatenate([dst, zpad])
    dsts_p = jnp.concatenate([dst, jnp.full((pad,), DUMMY, jnp.int32)])
    idx_all = (jnp.stack([src_p, dstg_p, dsts_p])
               .reshape(3, NT, BLOCKS, B).transpose(1, 2, 0, 3))

    f32 = jnp.float32

    xl0, xl1, xr0, xr1, xc = pl.pallas_call(
        _k1_body,
        grid=(N // _RB,),
        in_specs=[pl.BlockSpec((_RB, D), lambda i: (i, 0)),
                  _full_spec((D, 2 * C)), _full_spec((D, 2 * C)),
                  _full_spec((D, OUT))],
        out_specs=[pl.BlockSpec((_RB, C), lambda i: (i, 0))] * 4
        + [pl.BlockSpec((_RB, OUT), lambda i: (i, 0))],
        out_shape=[jax.ShapeDtypeStruct((N, C), f32)] * 4
        + [jax.ShapeDtypeStruct((N, OUT), f32)],
    )(x, W1_l, W1_r, Wc[D:])

    _gat_pass = _build_gat_pass()
    acc_h0, den_h0 = _gat_pass(xl0, xr0, att1[0], idx_all)
    acc_h1, den_h1 = _gat_pass(xl1, xr1, att1[1], idx_all)

    acc_spec = pl.BlockSpec((NC, _RB2, C), lambda i: (0, i, 0))
    den_spec = pl.BlockSpec((NT, _RB2), lambda i: (0, i))

    xl2f, xr2f = pl.pallas_call(
        _k2_body,
        grid=(ACC_ROWS // _RB2,),
        in_specs=[acc_spec, acc_spec, den_spec, den_spec,
                  _full_spec((1, 2 * C)), _full_spec((2 * C, C)),
                  _full_spec((2 * C, C))],
        out_specs=[pl.BlockSpec((_RB2, C), lambda i: (i, 0))] * 2,
        out_shape=[jax.ShapeDtypeStruct((ACC_ROWS, C), f32)] * 2,
    )(acc_h0, acc_h1, den_h0, den_h1, b1.reshape(1, 2 * C), W2_l, W2_r)

    acc2, den2 = _gat_pass(xl2f[:N], xr2f[:N], att2[0], idx_all)

    xc_pad = jnp.pad(xc, ((0, ACC_ROWS - N), (0, 0)))
    out_pad = pl.pallas_call(
        _k3_body,
        grid=(ACC_ROWS // _RB2,),
        in_specs=[acc_spec, den_spec, _full_spec((1, C)),
                  _full_spec((C, OUT)),
                  pl.BlockSpec((_RB2, OUT), lambda i: (i, 0)),
                  _full_spec((1, OUT))],
        out_specs=pl.BlockSpec((_RB2, OUT), lambda i: (i, 0)),
        out_shape=jax.ShapeDtypeStruct((ACC_ROWS, OUT), f32),
    )(acc2, den2, b2.reshape(1, C), Wc[:D], xc_pad, bc.reshape(1, OUT))

    return out_pad[:N]
